# baseline (speedup 1.0000x reference)
import functools

import jax
import jax.numpy as jnp
from jax import lax
from jax.experimental import pallas as pl
from jax.experimental.pallas import tpu as pltpu
from jax.experimental.pallas import tpu_sc as plsc

BATCH = 4096
SEQ = 200
EMBED = 32
TOTAL = BATCH * SEQ

NUM_CORES = 2
NUM_SUBCORES = 16
NW = NUM_CORES * NUM_SUBCORES
PER_WORKER = TOTAL // NW
NB = 4
CHUNK = 640
NCHUNK = PER_WORKER // CHUNK


def _emb_body(idx_hbm, table_hbm, out_hbm, idx_v, *scr):
    rows = scr[:NB]
    gsem = scr[NB:2 * NB]
    ssem = scr[2 * NB:3 * NB]

    wid = lax.axis_index("s") * NUM_CORES + lax.axis_index("c")
    base = wid * PER_WORKER

    pltpu.sync_copy(idx_hbm.at[pl.ds(base, PER_WORKER)], idx_v)

    def gdesc(i, b):
        return pltpu.make_async_copy(
            table_hbm.at[idx_v.at[pl.ds(i * CHUNK, CHUNK)]], rows[b], gsem[b])

    def sdesc(i, b):
        return pltpu.make_async_copy(
            rows[b], out_hbm.at[pl.ds(base + i * CHUNK, CHUNK)], ssem[b])

    for k in range(NB - 1):
        gdesc(k, k).start()

    @pl.loop(0, NCHUNK, step=NB)
    def _(i):
        for b in range(NB):
            j = i + b
            gdesc(0, b).wait()
            sdesc(j, b).start()
            nxt = j + NB - 1
            pb = (b + NB - 1) % NB

            @pl.when(jnp.logical_and(nxt >= NB, nxt <= NCHUNK - 1))
            def _():
                sdesc(0, pb).wait()
                gdesc(nxt, pb).start()

            @pl.when(jnp.logical_and(nxt < NB, nxt <= NCHUNK - 1))
            def _():
                gdesc(nxt, pb).start()

    for b in range(NB):
        sdesc(0, b).wait()


@jax.jit
def _embedding_lookup(x_flat, table):
    mesh = plsc.VectorSubcoreMesh(core_axis_name="c", subcore_axis_name="s")
    kern = functools.partial(
        pl.kernel,
        mesh=mesh,
        out_type=jax.ShapeDtypeStruct((TOTAL, EMBED), jnp.float32),
        scratch_types=(
            [pltpu.VMEM((PER_WORKER,), jnp.int32)]
            + [pltpu.VMEM((CHUNK, EMBED), jnp.float32)] * NB
            + [pltpu.SemaphoreType.DMA] * (2 * NB)
        ),
        compiler_params=pltpu.CompilerParams(use_tc_tiling_on_sc=False),
    )(_emb_body)
    return kern(x_flat, table)


def kernel(x, W_embed):
    xt = (x.astype(jnp.int32)
          .reshape(BATCH // 128, 128, SEQ // 8, 8)
          .transpose(2, 0, 3, 1)
          .reshape(TOTAL))
    out_p = _embedding_lookup(xt, W_embed)
    return (out_p.reshape(SEQ // 8, BATCH // 128, 8, 128, EMBED)
            .transpose(1, 3, 0, 2, 4)
            .reshape(BATCH, SEQ, EMBED))


# speedup vs baseline: 1.0245x; 1.0245x over previous
import functools

import jax
import jax.numpy as jnp
from jax import lax
from jax.experimental import pallas as pl
from jax.experimental.pallas import tpu as pltpu
from jax.experimental.pallas import tpu_sc as plsc

BATCH = 4096
SEQ = 200
EMBED = 32
TOTAL = BATCH * SEQ

NUM_CORES = 2
NUM_SUBCORES = 16
NW = NUM_CORES * NUM_SUBCORES
CHUNK = 512
RS_PER_CHUNK = CHUNK // 128
NCHUNK_TOTAL = TOTAL // CHUNK
PER_WORKER = NCHUNK_TOTAL // NW


def _emb_body(idx_hbm, table_hbm, out_hbm, *scr):
    idx_v = scr[0:2]
    g = scr[2:4]
    tbuf = scr[4:6]
    gsem = scr[6:8]
    ssem = scr[8:10]

    wid = lax.axis_index("s") * NUM_CORES + lax.axis_index("c")
    c0 = wid * PER_WORKER

    iota = lax.iota(jnp.int32, 16)
    te0 = iota // 8
    te1 = te0 + 2
    re_v = iota % 8

    def gstart(j, b):
        pltpu.sync_copy(idx_hbm.at[pl.ds((c0 + j) * CHUNK, CHUNK)], idx_v[b])
        pltpu.make_async_copy(table_hbm.at[idx_v[b]], g[b], gsem[b]).start()

    def gwait(b):
        pltpu.make_async_copy(table_hbm.at[idx_v[b]], g[b], gsem[b]).wait()

    def sdesc(j, b):
        cj = c0 + j
        p = cj // 2
        h = cj % 2
        ts = p // 32
        tb = p % 32
        s0 = ts * 8 + h * 4
        return pltpu.make_async_copy(
            tbuf[b], out_hbm.at[pl.ds(s0, RS_PER_CHUNK), :, tb], ssem[b])

    def transpose(b):
        @pl.loop(0, CHUNK)
        def _(r):
            rs_l = r // 128
            rb = r % 128
            v0 = g[b][r, pl.ds(0, 16)]
            v1 = g[b][r, pl.ds(16, 16)]
            rs_f = jnp.full((16,), rs_l, jnp.int32)
            rb_f = jnp.full((16,), rb, jnp.int32)
            plsc.store_scatter(tbuf[b], [rs_f, te0, re_v, rb_f], v0)
            plsc.store_scatter(tbuf[b], [rs_f, te1, re_v, rb_f], v1)

    gstart(0, 0)

    @pl.loop(0, PER_WORKER, step=2)
    def _(i):
        for b in range(2):
            j = i + b
            ob = 1 - b

            @pl.when(j + 1 <= PER_WORKER - 1)
            def _():
                gstart(j + 1, ob)

            gwait(b)

            @pl.when(j >= 2)
            def _():
                sdesc(0, b).wait()

            transpose(b)
            sdesc(j, b).start()

    for b in range(2):
        sdesc(0, b).wait()


@jax.jit
def _embedding_lookup(x_flat, table):
    mesh = plsc.VectorSubcoreMesh(core_axis_name="c", subcore_axis_name="s")
    kern = functools.partial(
        pl.kernel,
        mesh=mesh,
        out_type=jax.ShapeDtypeStruct((SEQ, 4, BATCH // 128, 8, 128),
                                      jnp.float32),
        scratch_types=(
            [pltpu.VMEM((CHUNK,), jnp.int32)] * 2
            + [pltpu.VMEM((CHUNK, EMBED), jnp.float32)] * 2
            + [pltpu.VMEM((RS_PER_CHUNK, 4, 8, 128), jnp.float32)] * 2
            + [pltpu.SemaphoreType.DMA] * 4
        ),
        compiler_params=pltpu.CompilerParams(use_tc_tiling_on_sc=False,
                                             needs_layout_passes=False),
    )(_emb_body)
    return kern(x_flat, table)


def kernel(x, W_embed):
    xt = (x.astype(jnp.int32)
          .reshape(BATCH // 128, 128, SEQ // 8, 8)
          .transpose(2, 0, 3, 1)
          .reshape(TOTAL))
    out5 = _embedding_lookup(xt, W_embed)
    return (out5.transpose(2, 4, 0, 1, 3)
            .reshape(BATCH, SEQ, EMBED))


# speedup vs baseline: 1.0252x; 1.0007x over previous
import functools

import jax
import jax.numpy as jnp
from jax import lax
from jax.experimental import pallas as pl
from jax.experimental.pallas import tpu as pltpu
from jax.experimental.pallas import tpu_sc as plsc

BATCH = 4096
SEQ = 200
EMBED = 32
TOTAL = BATCH * SEQ

NUM_CORES = 2
NUM_SUBCORES = 16
NW = NUM_CORES * NUM_SUBCORES
CHUNK = 512
RS_PER_CHUNK = CHUNK // 128
NCHUNK_TOTAL = TOTAL // CHUNK
PER_WORKER = NCHUNK_TOTAL // NW


def _emb_body(idx_hbm, table_hbm, out_hbm, *scr):
    idx_v = scr[0:2]
    g = scr[2:4]
    tbuf = scr[4:6]
    gsem = scr[6:8]
    ssem = scr[8:10]

    wid = lax.axis_index("s") * NUM_CORES + lax.axis_index("c")
    c0 = wid * PER_WORKER

    iota = lax.iota(jnp.int32, 16)
    te0 = iota // 8
    te1 = te0 + 2
    re_v = iota % 8

    def gstart(j, b):
        pltpu.sync_copy(idx_hbm.at[pl.ds((c0 + j) * CHUNK, CHUNK)], idx_v[b])
        pltpu.make_async_copy(table_hbm.at[idx_v[b]], g[b], gsem[b]).start()

    def gwait(b):
        pltpu.make_async_copy(table_hbm.at[idx_v[b]], g[b], gsem[b]).wait()

    def sdesc(j, b):
        cj = c0 + j
        p = cj // 2
        h = cj % 2
        ts = p // 32
        tb = p % 32
        s0 = ts * 8 + h * 4
        return pltpu.make_async_copy(
            tbuf[b], out_hbm.at[pl.ds(s0, RS_PER_CHUNK), :, tb], ssem[b])

    z16 = iota * 0

    def transpose(b):
        for rs_l in range(RS_PER_CHUNK):
            rs_f = jnp.full((16,), rs_l, jnp.int32)

            @pl.loop(0, 128, unroll=8)
            def _(rb):
                r = rs_l * 128 + rb
                v0 = g[b][r, pl.ds(0, 16)]
                v1 = g[b][r, pl.ds(16, 16)]
                rb_f = z16 + rb
                plsc.store_scatter(tbuf[b], [rs_f, te0, re_v, rb_f], v0)
                plsc.store_scatter(tbuf[b], [rs_f, te1, re_v, rb_f], v1)

    gstart(0, 0)

    @pl.loop(0, PER_WORKER, step=2)
    def _(i):
        for b in range(2):
            j = i + b
            ob = 1 - b

            @pl.when(j + 1 <= PER_WORKER - 1)
            def _():
                gstart(j + 1, ob)

            gwait(b)

            @pl.when(j >= 2)
            def _():
                sdesc(0, b).wait()

            transpose(b)
            sdesc(j, b).start()

    for b in range(2):
        sdesc(0, b).wait()


@jax.jit
def _embedding_lookup(x_flat, table):
    mesh = plsc.VectorSubcoreMesh(core_axis_name="c", subcore_axis_name="s")
    kern = functools.partial(
        pl.kernel,
        mesh=mesh,
        out_type=jax.ShapeDtypeStruct((SEQ, 4, BATCH // 128, 8, 128),
                                      jnp.float32),
        scratch_types=(
            [pltpu.VMEM((CHUNK,), jnp.int32)] * 2
            + [pltpu.VMEM((CHUNK, EMBED), jnp.float32)] * 2
            + [pltpu.VMEM((RS_PER_CHUNK, 4, 8, 128), jnp.float32)] * 2
            + [pltpu.SemaphoreType.DMA] * 4
        ),
        compiler_params=pltpu.CompilerParams(use_tc_tiling_on_sc=False,
                                             needs_layout_passes=False),
    )(_emb_body)
    return kern(x_flat, table)


def kernel(x, W_embed):
    xt = (x.astype(jnp.int32)
          .reshape(BATCH // 128, 128, SEQ // 8, 8)
          .transpose(2, 0, 3, 1)
          .reshape(TOTAL))
    out5 = _embedding_lookup(xt, W_embed)
    return (out5.transpose(2, 4, 0, 1, 3)
            .reshape(BATCH, SEQ, EMBED))


# speedup vs baseline: 1.5517x; 1.5135x over previous
import functools

import jax
import jax.numpy as jnp
from jax import lax
from jax.experimental import pallas as pl
from jax.experimental.pallas import tpu as pltpu
from jax.experimental.pallas import tpu_sc as plsc

BATCH = 4096
SEQ = 200
EMBED = 32
TOTAL = BATCH * SEQ

NUM_CORES = 2
NUM_SUBCORES = 16
NW = NUM_CORES * NUM_SUBCORES
CHUNK = 512
RS_PER_CHUNK = CHUNK // 128
NCHUNK_TOTAL = TOTAL // CHUNK
PER_WORKER = NCHUNK_TOTAL // NW


def _emb_body(idx_hbm, table_hbm, out_hbm, *scr):
    idx_v = scr[0:2]
    g = scr[2:4]
    tbuf = scr[4:6]
    gsem = scr[6:8]
    ssem = scr[8:10]

    wid = lax.axis_index("s") * NUM_CORES + lax.axis_index("c")
    c0 = wid * PER_WORKER

    iota = lax.iota(jnp.int32, 16)
    te0 = iota // 8
    te1 = te0 + 2
    re_v = iota % 8

    def gstart(j, b):
        pltpu.sync_copy(idx_hbm.at[pl.ds((c0 + j) * CHUNK, CHUNK)], idx_v[b])
        pltpu.make_async_copy(table_hbm.at[idx_v[b]], g[b], gsem[b]).start()

    def gwait(b):
        pltpu.make_async_copy(table_hbm.at[idx_v[b]], g[b], gsem[b]).wait()

    def sdesc(j, b):
        cj = c0 + j
        p = cj // 2
        h = cj % 2
        ts = p // 32
        tb = p % 32
        s0 = ts * 8 + h * 4
        return pltpu.make_async_copy(
            tbuf[b].at[:, :, :, pl.ds(0, 128)],
            out_hbm.at[pl.ds(s0, RS_PER_CHUNK), :, tb], ssem[b])

    z16 = iota * 0

    def transpose(b):
        for rs_l in range(RS_PER_CHUNK):
            rs_f = jnp.full((16,), rs_l, jnp.int32)

            @pl.loop(0, 128, unroll=8)
            def _(rb):
                r = rs_l * 128 + rb
                v0 = g[b][r, pl.ds(0, 16)]
                v1 = g[b][r, pl.ds(16, 16)]
                rb_f = z16 + rb
                plsc.store_scatter(tbuf[b], [rs_f, te0, re_v, rb_f], v0)
                plsc.store_scatter(tbuf[b], [rs_f, te1, re_v, rb_f], v1)

    gstart(0, 0)

    @pl.loop(0, PER_WORKER, step=2)
    def _(i):
        for b in range(2):
            j = i + b
            ob = 1 - b

            @pl.when(j + 1 <= PER_WORKER - 1)
            def _():
                gstart(j + 1, ob)

            gwait(b)

            @pl.when(j >= 2)
            def _():
                sdesc(0, b).wait()

            transpose(b)
            sdesc(j, b).start()

    for b in range(2):
        sdesc(0, b).wait()


@jax.jit
def _embedding_lookup(x_flat, table):
    mesh = plsc.VectorSubcoreMesh(core_axis_name="c", subcore_axis_name="s")
    kern = functools.partial(
        pl.kernel,
        mesh=mesh,
        out_type=jax.ShapeDtypeStruct((SEQ, 4, BATCH // 128, 8, 128),
                                      jnp.float32),
        scratch_types=(
            [pltpu.VMEM((CHUNK,), jnp.int32)] * 2
            + [pltpu.VMEM((CHUNK, EMBED), jnp.float32)] * 2
            + [pltpu.VMEM((RS_PER_CHUNK, 4, 8, 129), jnp.float32)] * 2
            + [pltpu.SemaphoreType.DMA] * 4
        ),
        compiler_params=pltpu.CompilerParams(use_tc_tiling_on_sc=False,
                                             needs_layout_passes=False),
    )(_emb_body)
    return kern(x_flat, table)


def kernel(x, W_embed):
    xt = (x.astype(jnp.int32)
          .reshape(BATCH // 128, 128, SEQ // 8, 8)
          .transpose(2, 0, 3, 1)
          .reshape(TOTAL))
    out5 = _embedding_lookup(xt, W_embed)
    return (out5.transpose(2, 4, 0, 1, 3)
            .reshape(BATCH, SEQ, EMBED))
